# baseline (speedup 1.0000x reference)
import jax
import jax.numpy as jnp
from jax import lax
from jax.experimental import pallas as pl
from jax.experimental.pallas import tpu as pltpu
from jax.experimental.pallas import tpu_sc as plsc

B, C, H, W = 4, 96, 384, 384
L = 384
NC, NS = 2, 16
NCH = C // 8
ROWS = 8
NCHUNK = H // ROWS
NV = ROWS * W // 16
LP = L + 8
ACC = NCH * LP


def _body(inp_hbm, idx_hbm, cnt_hbm, out_hbm,
          ids_v, in_v, acc_v, cnt_v, sem0, sem1):
    wid = lax.axis_index("c") * NS + lax.axis_index("s")
    b = wid // 8
    cg = wid % 8
    row0 = b * C + cg * NCH
    sems = (sem0, sem1)

    def zero(i, carry):
        acc_v[pl.ds(i * 16, 16)] = jnp.zeros((16,), jnp.float32)
        return carry
    lax.fori_loop(0, ACC // 16, zero, 0)

    pltpu.sync_copy(cnt_hbm.at[pl.ds(b * L, L)], cnt_v)

    def _descs(t, s):
        ci = lax.rem(t + cg * (NCHUNK // 8), NCHUNK)
        h0 = ci * ROWS
        return [
            pltpu.make_async_copy(
                idx_hbm.at[b, pl.ds(h0, ROWS), :], ids_v.at[s], sems[s]),
            pltpu.make_async_copy(
                inp_hbm.at[b, pl.ds(cg * NCH, NCH), pl.ds(h0, ROWS), :],
                in_v.at[s], sems[s]),
        ]

    def fire(t, s):
        for d in _descs(t, s):
            d.start()

    def wait(t, s):
        for d in _descs(t, s):
            d.wait()

    def compute(s):
        @plsc.parallel_loop(0, NV, unroll=16)
        def vec(i):
            r = i // (W // 16)
            cw = (i % (W // 16)) * 16
            ivr = ids_v[s, r, pl.ds(cw, 16)]
            mv = ivr >= 0
            iv = jnp.bitwise_and(ivr, 0x1FF)
            for j in range(NCH):
                x = in_v[s, j, r, pl.ds(cw, 16)]
                plsc.addupdate_scatter(
                    acc_v.at[pl.ds(j * LP, L)], [iv], x, mask=mv)

    fire(0, 0)
    fire(1, 1)

    def outer(g, carry):
        for s in range(2):
            t = g * 2 + s
            wait(t, s)
            compute(s)

            @pl.when(t + 2 < NCHUNK)
            def _():
                fire(t + 2, s)
        return carry
    lax.fori_loop(0, NCHUNK // 2, outer, 0)

    def fin(i, carry):
        j = i // (L // 16)
        m = lax.rem(i, L // 16) * 16
        a = acc_v[pl.ds(j * LP + m, 16)]
        cnt = cnt_v[pl.ds(m, 16)]
        acc_v[pl.ds(i * 16, 16)] = a / cnt
        return carry
    lax.fori_loop(0, NCH * (L // 16), fin, 0)

    pltpu.sync_copy(acc_v.at[pl.ds(0, NCH * L)],
                    out_hbm.at[pl.ds(row0 * L, NCH * L)])


@jax.jit
def _run(inp, idx, cnt):
    mesh = plsc.VectorSubcoreMesh(core_axis_name="c", subcore_axis_name="s")
    return pl.kernel(
        _body,
        out_type=jax.ShapeDtypeStruct((B * C * L,), jnp.float32),
        mesh=mesh,
        compiler_params=pltpu.CompilerParams(needs_layout_passes=False),
        scratch_types=[
            pltpu.VMEM((2, ROWS, W), jnp.int32),
            pltpu.VMEM((2, NCH, ROWS, W), jnp.float32),
            pltpu.VMEM((ACC,), jnp.float32),
            pltpu.VMEM((L,), jnp.float32),
            pltpu.SemaphoreType.DMA,
            pltpu.SemaphoreType.DMA,
        ],
    )(inp, idx, cnt)


def kernel(input, output_count, indmap, valid_maps):
    idx = indmap.astype(jnp.int32)
    vld = valid_maps.astype(jnp.int32)
    idx = jnp.where(vld > 0, idx, idx | jnp.int32(-2147483648))
    out = _run(input, idx, output_count.reshape(B * L))
    return out.reshape(B, C, L)


# speedup vs baseline: 1.1113x; 1.1113x over previous
import jax
import jax.numpy as jnp
from jax import lax
from jax.experimental import pallas as pl
from jax.experimental.pallas import tpu as pltpu
from jax.experimental.pallas import tpu_sc as plsc

B, C, H, W = 4, 96, 384, 384
L = 384
NC, NS = 2, 16
NCH = C // 8
ROWS = 8
NCHUNK = H // ROWS
NV = ROWS * W // 16
LP = L + 8
ACC = NCH * LP


def _body(inp_hbm, idx_hbm, cnt_hbm, out_hbm,
          ids_v, in_v, acc_v, cnt_v, sem0, sem1):
    wid = lax.axis_index("c") * NS + lax.axis_index("s")
    b = wid // 8
    cg = wid % 8
    row0 = b * C + cg * NCH
    sems = (sem0, sem1)

    def zero(i, carry):
        acc_v[pl.ds(i * 16, 16)] = jnp.zeros((16,), jnp.float32)
        return carry
    lax.fori_loop(0, ACC // 16, zero, 0)

    pltpu.sync_copy(cnt_hbm.at[pl.ds(b * L, L)], cnt_v)

    def _descs(t, s):
        ci = lax.rem(t + cg * (NCHUNK // 8), NCHUNK)
        h0 = ci * ROWS
        return [
            pltpu.make_async_copy(
                idx_hbm.at[b, pl.ds(h0, ROWS), :], ids_v.at[s], sems[s]),
            pltpu.make_async_copy(
                inp_hbm.at[b, pl.ds(cg * NCH, NCH), pl.ds(h0, ROWS), :],
                in_v.at[s], sems[s]),
        ]

    def fire(t, s):
        for d in _descs(t, s):
            d.start()

    def wait(t, s):
        for d in _descs(t, s):
            d.wait()

    def compute(s):
        @plsc.parallel_loop(0, NV, unroll=8)
        def vec(i):
            r = i // (W // 16)
            cw = (i % (W // 16)) * 16
            ivr = ids_v[s, r, pl.ds(cw, 16)]
            mv = ivr >= 0
            iv = jnp.bitwise_and(ivr, 0x1FF)
            for j in range(NCH):
                x = in_v[s, j, r, pl.ds(cw, 16)]
                plsc.addupdate_scatter(
                    acc_v.at[pl.ds(j * LP, L)], [iv], x, mask=mv)

    fire(0, 0)
    fire(1, 1)

    def outer(g, carry):
        for s in range(2):
            t = g * 2 + s
            wait(t, s)
            compute(s)

            @pl.when(t + 2 < NCHUNK)
            def _():
                fire(t + 2, s)
        return carry
    lax.fori_loop(0, NCHUNK // 2, outer, 0)

    def fin(i, carry):
        j = i // (L // 16)
        m = lax.rem(i, L // 16) * 16
        a = acc_v[pl.ds(j * LP + m, 16)]
        cnt = cnt_v[pl.ds(m, 16)]
        acc_v[pl.ds(i * 16, 16)] = a / cnt
        return carry
    lax.fori_loop(0, NCH * (L // 16), fin, 0)

    pltpu.sync_copy(acc_v.at[pl.ds(0, NCH * L)],
                    out_hbm.at[pl.ds(row0 * L, NCH * L)])


@jax.jit
def _run(inp, idx, cnt):
    mesh = plsc.VectorSubcoreMesh(core_axis_name="c", subcore_axis_name="s")
    return pl.kernel(
        _body,
        out_type=jax.ShapeDtypeStruct((B * C * L,), jnp.float32),
        mesh=mesh,
        compiler_params=pltpu.CompilerParams(needs_layout_passes=False),
        scratch_types=[
            pltpu.VMEM((2, ROWS, W), jnp.int32),
            pltpu.VMEM((2, NCH, ROWS, W), jnp.float32),
            pltpu.VMEM((ACC,), jnp.float32),
            pltpu.VMEM((L,), jnp.float32),
            pltpu.SemaphoreType.DMA,
            pltpu.SemaphoreType.DMA,
        ],
    )(inp, idx, cnt)


def kernel(input, output_count, indmap, valid_maps):
    idx = indmap.astype(jnp.int32)
    vld = valid_maps.astype(jnp.int32)
    idx = jnp.where(vld > 0, idx, idx | jnp.int32(-2147483648))
    out = _run(input, idx, output_count.reshape(B * L))
    return out.reshape(B, C, L)
